# baseline (speedup 1.0000x reference)
import functools

import jax
import jax.numpy as jnp
from jax.experimental import pallas as pl
from jax.experimental.pallas import tpu as pltpu
from jax.experimental.pallas import tpu_sc as plsc

N = 1024
C = 512
K = 16
PD = 128
TILE = 128

_INTERPRET = False


def _prep_body(k2_ref, k3_ref, mrows_ref, w2d_ref, b2d_ref, w3d_ref, b3d_ref,
               ph_ref, f2_ref, f3_ref):
    lane = jax.lax.broadcasted_iota(jnp.int32, (N, PD), 1)
    x = k2_ref[:, 0:1]
    y = k2_ref[:, 1:2]
    r0 = mrows_ref[0:1, :]
    r1 = mrows_ref[1:2, :]
    r2 = mrows_ref[2:3, :]
    bea = y * r0 + x * r1 + r2
    n2 = jnp.sqrt(jnp.sum(bea * bea, axis=1, keepdims=True))
    p2 = bea / jnp.maximum(n2, 1e-12)
    z = k3_ref[:, 2:3]
    c0 = k3_ref[:, 0:1] / z
    c1 = k3_ref[:, 1:2] / z
    w = jnp.where(lane == 0, c0, jnp.where(lane == 1, c1,
                  jnp.where(lane == 2, 1.0, 0.0)))
    n3 = jnp.sqrt(jnp.sum(w * w, axis=1, keepdims=True))
    p3 = w / jnp.maximum(n3, 1e-12)
    f2_ref[...] = jnp.dot(p2, w2d_ref[...],
                          preferred_element_type=jnp.float32) + b2d_ref[0:1, :]
    f3_ref[...] = jnp.dot(p3, w3d_ref[...],
                          preferred_element_type=jnp.float32) + b3d_ref[0:1, :]
    ph_ref[0:N] = jnp.where(lane == 3, 1.0, p2)
    ph_ref[N:2 * N] = jnp.where(lane == 3, 1.0, p3)


def _prep(k2, k3, mrows, w2d_pad, b2d, w3d_pad, b3d):
    f32 = jnp.float32
    return pl.pallas_call(
        _prep_body,
        out_shape=(jax.ShapeDtypeStruct((2 * N, PD), f32),
                   jax.ShapeDtypeStruct((N, C), f32),
                   jax.ShapeDtypeStruct((N, C), f32)),
        interpret=_INTERPRET,
    )(k2, k3, mrows, w2d_pad, b2d, w3d_pad, b3d)


def _knn_body(ph_ref, phT_ref, idx_ref):
    g = jnp.dot(ph_ref[...], phT_ref[0], preferred_element_type=jnp.float32)
    cols = jax.lax.broadcasted_iota(jnp.int32, (TILE, N), 1)
    for t in range(K):
        m = jnp.max(g, axis=1, keepdims=True)
        cand = jnp.where(g >= m, cols, jnp.int32(N))
        amin = jnp.min(cand, axis=1)
        idx_ref[0, t, :] = amin
        g = jnp.where(cols == amin[:, None], jnp.float32(-3.0e38), g)


def _knn(ph, phT):
    nt = N // TILE
    return pl.pallas_call(
        _knn_body,
        grid=(2 * nt,),
        in_specs=[pl.BlockSpec((TILE, PD), lambda i: (i, 0)),
                  pl.BlockSpec((1, PD, N), lambda i: (i // (N // TILE), 0, 0))],
        out_specs=pl.BlockSpec((1, K, TILE), lambda i: (i // (N // TILE), 0,
                                                        i % (N // TILE))),
        out_shape=jax.ShapeDtypeStruct((2, K, N), jnp.int32),
        compiler_params=pltpu.CompilerParams(
            dimension_semantics=("parallel",)),
        interpret=_INTERPRET,
    )(ph, phT)


def _rne_bf16_bits(x):
    b = jax.lax.bitcast_convert_type(x, jnp.int32)
    return (b + 0x7FFF + ((b >> 16) & 1)) & jnp.int32(-65536)


def _qkv_body(f_ref, wq_ref, wkv_ref, q_ref, kvp_ref):
    f = f_ref[...].astype(jnp.bfloat16)
    q_ref[...] = jnp.dot(f, wq_ref[...].astype(jnp.bfloat16),
                         preferred_element_type=jnp.float32)
    kv = jnp.dot(f, wkv_ref[...].astype(jnp.bfloat16),
                 preferred_element_type=jnp.float32)
    kb = _rne_bf16_bits(kv[:, 0:C])
    vb = _rne_bf16_bits(kv[:, C:2 * C])
    kvp_ref[...] = kb | ((vb >> 16) & 0xFFFF)


def _qkv(f, wq, wkv):
    return pl.pallas_call(
        _qkv_body,
        out_shape=(jax.ShapeDtypeStruct((N, C), jnp.float32),
                   jax.ShapeDtypeStruct((N, C), jnp.int32)),
        interpret=_INTERPRET,
    )(f, wq, wkv)


def _gather_rows(table, idx_flat, d, chunk, dtype=jnp.float32):
    b_total = idx_flat.shape[0]
    info = plsc.get_sparse_core_info()
    nw = info.num_cores * info.num_subcores
    b_per_w = b_total // nw
    n_chunks = b_per_w // chunk
    mesh = plsc.VectorSubcoreMesh(core_axis_name="c", subcore_axis_name="s")

    @functools.partial(
        pl.kernel,
        out_type=jax.ShapeDtypeStruct((b_total, d), dtype),
        mesh=mesh,
        scratch_types=[
            pltpu.VMEM((b_per_w,), jnp.int32),
            pltpu.VMEM((chunk, d), dtype),
            pltpu.VMEM((chunk, d), dtype),
            pltpu.SemaphoreType.DMA,
            pltpu.SemaphoreType.DMA,
            pltpu.SemaphoreType.DMA,
            pltpu.SemaphoreType.DMA,
        ],
    )
    def kern(table_hbm, idx_hbm, out_hbm, idx_v, buf0, buf1, gs0, gs1, ss0, ss1):
        wid = jax.lax.axis_index("s") * info.num_cores + jax.lax.axis_index("c")
        base = wid * b_per_w
        pltpu.sync_copy(idx_hbm.at[pl.ds(base, b_per_w)], idx_v)
        bufs, gsems, ssems = (buf0, buf1), (gs0, gs1), (ss0, ss1)

        def gstart(ci):
            return pltpu.async_copy(
                table_hbm.at[idx_v.at[pl.ds(ci * chunk, chunk)]],
                bufs[ci % 2], gsems[ci % 2])

        def sstart(ci):
            return pltpu.async_copy(
                bufs[ci % 2], out_hbm.at[pl.ds(base + ci * chunk, chunk)],
                ssems[ci % 2])

        g = {0: gstart(0)}
        s = {}
        for ci in range(n_chunks):
            if ci + 1 < n_chunks:
                if ci - 1 >= 0:
                    s[ci - 1].wait()
                g[ci + 1] = gstart(ci + 1)
            g[ci].wait()
            s[ci] = sstart(ci)
        if n_chunks >= 2:
            s[n_chunks - 2].wait()
        s[n_chunks - 1].wait()

    return kern(table, idx_flat)


def _attn_body(q_ref, f_ref, ph_ref, kvn_ref, pn_ref, wp_ref, wa_ref, out_ref):
    q = q_ref[...]
    ph = ph_ref[...].astype(jnp.bfloat16)
    wp = wp_ref[...].astype(jnp.bfloat16)
    wa = wa_ref[...].astype(jnp.bfloat16)
    s = jnp.zeros((TILE, C), jnp.float32)
    acc = jnp.zeros((TILE, C), jnp.float32)
    for j in range(K):
        rel = ph - pn_ref[0, j].astype(jnp.bfloat16)
        pe = jnp.dot(rel, wp, preferred_element_type=jnp.float32)
        kn = jax.lax.bitcast_convert_type(
            kvn_ref[j] & jnp.int32(-65536), jnp.float32)
        t = jnp.maximum(q - kn + pe, 0.0)
        l = jnp.dot(t.astype(jnp.bfloat16), wa,
                    preferred_element_type=jnp.float32)
        e = jnp.exp(l)
        s = s + e
        vn = jax.lax.bitcast_convert_type(kvn_ref[j] << 16, jnp.float32)
        acc = acc + e * (vn + pe)
    out_ref[...] = f_ref[...] + acc / s


def _attn(q, f, ph, kvn, pn4, cloud, wp_pad, wa):
    kvn3 = kvn.reshape(K, N, C)
    nt = N // TILE
    return pl.pallas_call(
        _attn_body,
        grid=(nt,),
        in_specs=[
            pl.BlockSpec((TILE, C), lambda i: (i, 0)),
            pl.BlockSpec((TILE, C), lambda i: (i, 0)),
            pl.BlockSpec((TILE, PD),
                         lambda i: (cloud * (N // TILE) + i, 0)),
            pl.BlockSpec((K, TILE, C), lambda i: (0, i, 0)),
            pl.BlockSpec((1, K, TILE, PD),
                         lambda i: (cloud, 0, i, 0)),
            pl.BlockSpec((PD, C), lambda i: (0, 0)),
            pl.BlockSpec((C, C), lambda i: (0, 0)),
        ],
        out_specs=pl.BlockSpec((TILE, C), lambda i: (i, 0)),
        out_shape=jax.ShapeDtypeStruct((N, C), jnp.float32),
        compiler_params=pltpu.CompilerParams(
            dimension_semantics=("parallel",)),
        interpret=_INTERPRET,
    )(q, f, ph, kvn3, pn4, wp_pad, wa)


def _pool_body(f2_ref, f3_ref, w1_ref, w2_ref, b_ref, out_ref):
    m2 = jnp.sum(f2_ref[...], axis=0, keepdims=True) * (1.0 / N)
    m3 = jnp.sum(f3_ref[...], axis=0, keepdims=True) * (1.0 / N)
    out_ref[...] = (jnp.dot(m2, w1_ref[...], preferred_element_type=jnp.float32)
                    + jnp.dot(m3, w2_ref[...], preferred_element_type=jnp.float32)
                    + b_ref[...])


def _pool(f2, f3, w1_pad, w2_pad, b_pad):
    return pl.pallas_call(
        _pool_body,
        out_shape=jax.ShapeDtypeStruct((1, 128), jnp.float32),
        interpret=_INTERPRET,
    )(f2, f3, w1_pad, w2_pad, b_pad)


def _proj_body(ph_ref, a_ref, out_ref):
    y = jnp.dot(ph_ref[...], a_ref[...], preferred_element_type=jnp.float32)
    n = jnp.sqrt(jnp.sum(y * y, axis=1, keepdims=True))
    out_ref[...] = y / jnp.maximum(n, 1e-12)


def _proj(ph, amat):
    return pl.pallas_call(
        _proj_body,
        grid=(1,),
        in_specs=[pl.BlockSpec((N, PD), lambda i: (0, 0)),
                  pl.BlockSpec((PD, PD), lambda i: (0, 0))],
        out_specs=pl.BlockSpec((N, PD), lambda i: (0, 0)),
        out_shape=jax.ShapeDtypeStruct((N, PD), jnp.float32),
        interpret=_INTERPRET,
    )(ph, amat)


def _aa_to_rot(aa):
    theta = jnp.linalg.norm(aa) + 1e-12
    k = aa / theta
    z = jnp.zeros(())
    kx = jnp.stack([jnp.stack([z, -k[2], k[1]]),
                    jnp.stack([k[2], z, -k[0]]),
                    jnp.stack([-k[1], k[0], z])])
    return (jnp.eye(3) + jnp.sin(theta) * kx
            + (1.0 - jnp.cos(theta)) * (kx @ kx))


def _pad(x, rows, cols):
    return jnp.zeros((rows, cols), jnp.float32).at[:x.shape[0], :x.shape[1]].set(x)


def kernel(kpts_2d_pix, kpts_3d_pts, intrinsics, W2d_pre, b2d_pre, W3d_pre,
           b3d_pre, Wq1, Wk1, Wv1, Wa1, Wq2, Wk2, Wv2, Wa2, Wp1, Wp2,
           W_so, b_so):
    f32 = jnp.float32
    inv_intr = jnp.linalg.inv(intrinsics)
    mrows = _pad(inv_intr.T, 8, PD)
    w2d_pad = _pad(W2d_pre, PD, C)
    w3d_pad = _pad(W3d_pre, PD, C)
    b2d = jnp.broadcast_to(b2d_pre[None, :], (8, C)).astype(f32)
    b3d = jnp.broadcast_to(b3d_pre[None, :], (8, C)).astype(f32)

    ph, f2, f3 = _prep(kpts_2d_pix, kpts_3d_pts, mrows,
                       w2d_pad, b2d, w3d_pad, b3d)

    phT = ph.reshape(2, N, PD).transpose(0, 2, 1)
    idx = _knn(ph, phT)
    idx2f = idx[0].reshape(K * N)
    idx3f = idx[1].reshape(K * N)
    idxb = jnp.concatenate([idx2f, idx3f + N])

    pn4 = _gather_rows(ph, idxb, PD, 256).reshape(2, K, N, PD)

    wkv1 = jnp.concatenate([Wk1, Wv1], axis=1)
    wkv2 = jnp.concatenate([Wk2, Wv2], axis=1)
    wp1 = _pad(Wp1, PD, C)
    wp2 = _pad(Wp2, PD, C)

    for wq, wkv, wp, wa in ((Wq1, wkv1, wp1, Wa1), (Wq2, wkv2, wp2, Wa2)):
        q2, kvp2 = _qkv(f2, wq, wkv)
        kvn2 = _gather_rows(kvp2, idx2f, C, 64, jnp.int32)
        q3, kvp3 = _qkv(f3, wq, wkv)
        kvn3 = _gather_rows(kvp3, idx3f, C, 64, jnp.int32)
        f2 = _attn(q2, f2, ph, kvn2, pn4, 0, wp, wa)
        f3 = _attn(q3, f3, ph, kvn3, pn4, 1, wp, wa)

    wso1 = _pad(W_so[:C], C, 128)
    wso2 = _pad(W_so[C:], C, 128)
    bso = _pad(b_so[None, :], 1, 128)
    pose = _pool(f2, f3, wso1, wso2, bso)[0, :6]

    rot = _aa_to_rot(pose[0:3])
    amat = jnp.zeros((PD, PD), f32).at[0:3, 0:3].set(rot).at[3, 0:3].set(pose[3:6])
    out2d = _proj(ph, amat)

    kpts_2d_xyz = out2d[:, 0:3].T[None, :, :]
    kpts_3d_xyz = ph[N:2 * N, 0:3].T[None, :, :]
    return (kpts_2d_xyz, kpts_3d_xyz)


# speedup vs baseline: 1.0688x; 1.0688x over previous
import functools

import jax
import jax.numpy as jnp
from jax.experimental import pallas as pl
from jax.experimental.pallas import tpu as pltpu
from jax.experimental.pallas import tpu_sc as plsc

N = 1024
C = 512
K = 16
PD = 128
TILE = 128

_INTERPRET = False


def _prep_body(k2_ref, k3_ref, mrows_ref, w2d_ref, b2d_ref, w3d_ref, b3d_ref,
               ph2_ref, ph3_ref, f2_ref, f3_ref):
    lane = jax.lax.broadcasted_iota(jnp.int32, (N, PD), 1)
    x = k2_ref[:, 0:1]
    y = k2_ref[:, 1:2]
    r0 = mrows_ref[0:1, :]
    r1 = mrows_ref[1:2, :]
    r2 = mrows_ref[2:3, :]
    bea = y * r0 + x * r1 + r2
    n2 = jnp.sqrt(jnp.sum(bea * bea, axis=1, keepdims=True))
    p2 = bea / jnp.maximum(n2, 1e-12)
    z = k3_ref[:, 2:3]
    c0 = k3_ref[:, 0:1] / z
    c1 = k3_ref[:, 1:2] / z
    w = jnp.where(lane == 0, c0, jnp.where(lane == 1, c1,
                  jnp.where(lane == 2, 1.0, 0.0)))
    n3 = jnp.sqrt(jnp.sum(w * w, axis=1, keepdims=True))
    p3 = w / jnp.maximum(n3, 1e-12)
    f2_ref[...] = jnp.dot(p2, w2d_ref[...],
                          preferred_element_type=jnp.float32) + b2d_ref[0:1, :]
    f3_ref[...] = jnp.dot(p3, w3d_ref[...],
                          preferred_element_type=jnp.float32) + b3d_ref[0:1, :]
    ph2_ref[...] = jnp.where(lane == 3, 1.0, p2)
    ph3_ref[...] = jnp.where(lane == 3, 1.0, p3)


def _prep(k2, k3, mrows, w2d_pad, b2d, w3d_pad, b3d):
    f32 = jnp.float32
    return pl.pallas_call(
        _prep_body,
        out_shape=(jax.ShapeDtypeStruct((N, PD), f32),
                   jax.ShapeDtypeStruct((N, PD), f32),
                   jax.ShapeDtypeStruct((N, C), f32),
                   jax.ShapeDtypeStruct((N, C), f32)),
        interpret=_INTERPRET,
    )(k2, k3, mrows, w2d_pad, b2d, w3d_pad, b3d)


def _knn_body(ph_ref, phT_ref, idx_ref):
    g = jnp.dot(ph_ref[...], phT_ref[...], preferred_element_type=jnp.float32)
    cols = jax.lax.broadcasted_iota(jnp.int32, (TILE, N), 1)
    for t in range(K):
        m = jnp.max(g, axis=1, keepdims=True)
        cand = jnp.where(g >= m, cols, jnp.int32(N))
        amin = jnp.min(cand, axis=1)
        idx_ref[t, :] = amin
        g = jnp.where(cols == amin[:, None], jnp.float32(-3.0e38), g)


def _knn(ph, phT):
    return pl.pallas_call(
        _knn_body,
        grid=(N // TILE,),
        in_specs=[pl.BlockSpec((TILE, PD), lambda i: (i, 0)),
                  pl.BlockSpec((PD, N), lambda i: (0, 0))],
        out_specs=pl.BlockSpec((K, TILE), lambda i: (0, i)),
        out_shape=jax.ShapeDtypeStruct((K, N), jnp.int32),
        interpret=_INTERPRET,
    )(ph, phT)


def _rne_bf16_bits(x):
    b = jax.lax.bitcast_convert_type(x, jnp.int32)
    return (b + 0x7FFF + ((b >> 16) & 1)) & jnp.int32(-65536)


def _qkv_body(f_ref, wq_ref, wkv_ref, q_ref, kvp_ref):
    f = f_ref[...].astype(jnp.bfloat16)
    q_ref[...] = jnp.dot(f, wq_ref[...].astype(jnp.bfloat16),
                         preferred_element_type=jnp.float32)
    kv = jnp.dot(f, wkv_ref[...].astype(jnp.bfloat16),
                 preferred_element_type=jnp.float32)
    kb = _rne_bf16_bits(kv[:, 0:C])
    vb = _rne_bf16_bits(kv[:, C:2 * C])
    kvp_ref[...] = kb | ((vb >> 16) & 0xFFFF)


def _qkv(f, wq, wkv):
    return pl.pallas_call(
        _qkv_body,
        out_shape=(jax.ShapeDtypeStruct((N, C), jnp.float32),
                   jax.ShapeDtypeStruct((N, C), jnp.int32)),
        interpret=_INTERPRET,
    )(f, wq, wkv)


def _gather_rows(table, idx_flat, d, chunk, dtype=jnp.float32):
    b_total = idx_flat.shape[0]
    info = plsc.get_sparse_core_info()
    nw = info.num_cores * info.num_subcores
    b_per_w = b_total // nw
    n_chunks = b_per_w // chunk
    mesh = plsc.VectorSubcoreMesh(core_axis_name="c", subcore_axis_name="s")

    @functools.partial(
        pl.kernel,
        out_type=jax.ShapeDtypeStruct((b_total, d), dtype),
        mesh=mesh,
        scratch_types=[
            pltpu.VMEM((b_per_w,), jnp.int32),
            pltpu.VMEM((chunk, d), dtype),
            pltpu.VMEM((chunk, d), dtype),
            pltpu.SemaphoreType.DMA,
            pltpu.SemaphoreType.DMA,
            pltpu.SemaphoreType.DMA,
            pltpu.SemaphoreType.DMA,
        ],
    )
    def kern(table_hbm, idx_hbm, out_hbm, idx_v, buf0, buf1, gs0, gs1, ss0, ss1):
        wid = jax.lax.axis_index("s") * info.num_cores + jax.lax.axis_index("c")
        base = wid * b_per_w
        pltpu.sync_copy(idx_hbm.at[pl.ds(base, b_per_w)], idx_v)
        bufs, gsems, ssems = (buf0, buf1), (gs0, gs1), (ss0, ss1)

        def gstart(ci):
            return pltpu.async_copy(
                table_hbm.at[idx_v.at[pl.ds(ci * chunk, chunk)]],
                bufs[ci % 2], gsems[ci % 2])

        def sstart(ci):
            return pltpu.async_copy(
                bufs[ci % 2], out_hbm.at[pl.ds(base + ci * chunk, chunk)],
                ssems[ci % 2])

        g = {0: gstart(0)}
        s = {}
        for ci in range(n_chunks):
            if ci + 1 < n_chunks:
                if ci - 1 >= 0:
                    s[ci - 1].wait()
                g[ci + 1] = gstart(ci + 1)
            g[ci].wait()
            s[ci] = sstart(ci)
        if n_chunks >= 2:
            s[n_chunks - 2].wait()
        s[n_chunks - 1].wait()

    return kern(table, idx_flat)


def _attn_body(q_ref, f_ref, ph_ref, kvn_ref, pn_ref, wp_ref, wa_ref, out_ref):
    q = q_ref[...]
    ph = ph_ref[...].astype(jnp.bfloat16)
    wp = wp_ref[...].astype(jnp.bfloat16)
    wa = wa_ref[...].astype(jnp.bfloat16)
    s = jnp.zeros((TILE, C), jnp.float32)
    acc = jnp.zeros((TILE, C), jnp.float32)
    for j in range(K):
        rel = ph - pn_ref[j].astype(jnp.bfloat16)
        pe = jnp.dot(rel, wp, preferred_element_type=jnp.float32)
        kn = jax.lax.bitcast_convert_type(
            kvn_ref[j] & jnp.int32(-65536), jnp.float32)
        t = jnp.maximum(q - kn + pe, 0.0)
        l = jnp.dot(t.astype(jnp.bfloat16), wa,
                    preferred_element_type=jnp.float32)
        e = jnp.exp(l)
        s = s + e
        vn = jax.lax.bitcast_convert_type(kvn_ref[j] << 16, jnp.float32)
        acc = acc + e * (vn + pe)
    out_ref[...] = f_ref[...] + acc / s


def _attn(q, f, ph, kvn, pn, wp_pad, wa):
    kvn3 = kvn.reshape(K, N, C)
    pn3 = pn.reshape(K, N, PD)
    return pl.pallas_call(
        _attn_body,
        grid=(N // TILE,),
        in_specs=[
            pl.BlockSpec((TILE, C), lambda i: (i, 0)),
            pl.BlockSpec((TILE, C), lambda i: (i, 0)),
            pl.BlockSpec((TILE, PD), lambda i: (i, 0)),
            pl.BlockSpec((K, TILE, C), lambda i: (0, i, 0)),
            pl.BlockSpec((K, TILE, PD), lambda i: (0, i, 0)),
            pl.BlockSpec((PD, C), lambda i: (0, 0)),
            pl.BlockSpec((C, C), lambda i: (0, 0)),
        ],
        out_specs=pl.BlockSpec((TILE, C), lambda i: (i, 0)),
        out_shape=jax.ShapeDtypeStruct((N, C), jnp.float32),
        interpret=_INTERPRET,
    )(q, f, ph, kvn3, pn3, wp_pad, wa)


def _pool_body(f2_ref, f3_ref, w1_ref, w2_ref, b_ref, out_ref):
    m2 = jnp.sum(f2_ref[...], axis=0, keepdims=True) * (1.0 / N)
    m3 = jnp.sum(f3_ref[...], axis=0, keepdims=True) * (1.0 / N)
    out_ref[...] = (jnp.dot(m2, w1_ref[...], preferred_element_type=jnp.float32)
                    + jnp.dot(m3, w2_ref[...], preferred_element_type=jnp.float32)
                    + b_ref[...])


def _pool(f2, f3, w1_pad, w2_pad, b_pad):
    return pl.pallas_call(
        _pool_body,
        out_shape=jax.ShapeDtypeStruct((1, 128), jnp.float32),
        interpret=_INTERPRET,
    )(f2, f3, w1_pad, w2_pad, b_pad)


def _proj_body(ph_ref, a_ref, out_ref):
    y = jnp.dot(ph_ref[...], a_ref[...], preferred_element_type=jnp.float32)
    n = jnp.sqrt(jnp.sum(y * y, axis=1, keepdims=True))
    out_ref[...] = y / jnp.maximum(n, 1e-12)


def _proj(ph, amat):
    return pl.pallas_call(
        _proj_body,
        out_shape=jax.ShapeDtypeStruct((N, PD), jnp.float32),
        interpret=_INTERPRET,
    )(ph, amat)


def _aa_to_rot(aa):
    theta = jnp.linalg.norm(aa) + 1e-12
    k = aa / theta
    z = jnp.zeros(())
    kx = jnp.stack([jnp.stack([z, -k[2], k[1]]),
                    jnp.stack([k[2], z, -k[0]]),
                    jnp.stack([-k[1], k[0], z])])
    return (jnp.eye(3) + jnp.sin(theta) * kx
            + (1.0 - jnp.cos(theta)) * (kx @ kx))


def _pad(x, rows, cols):
    return jnp.zeros((rows, cols), jnp.float32).at[:x.shape[0], :x.shape[1]].set(x)


def kernel(kpts_2d_pix, kpts_3d_pts, intrinsics, W2d_pre, b2d_pre, W3d_pre,
           b3d_pre, Wq1, Wk1, Wv1, Wa1, Wq2, Wk2, Wv2, Wa2, Wp1, Wp2,
           W_so, b_so):
    f32 = jnp.float32
    inv_intr = jnp.linalg.inv(intrinsics)
    mrows = _pad(inv_intr.T, 8, PD)
    w2d_pad = _pad(W2d_pre, PD, C)
    w3d_pad = _pad(W3d_pre, PD, C)
    b2d = jnp.broadcast_to(b2d_pre[None, :], (8, C)).astype(f32)
    b3d = jnp.broadcast_to(b3d_pre[None, :], (8, C)).astype(f32)

    ph2, ph3, f2, f3 = _prep(kpts_2d_pix, kpts_3d_pts, mrows,
                             w2d_pad, b2d, w3d_pad, b3d)

    idx2 = _knn(ph2, ph2.T)
    idx3 = _knn(ph3, ph3.T)
    idx2f = idx2.reshape(K * N)
    idx3f = idx3.reshape(K * N)

    pn2 = _gather_rows(ph2, idx2f, PD, 256)
    pn3 = _gather_rows(ph3, idx3f, PD, 256)

    wkv1 = jnp.concatenate([Wk1, Wv1], axis=1)
    wkv2 = jnp.concatenate([Wk2, Wv2], axis=1)
    wp1 = _pad(Wp1, PD, C)
    wp2 = _pad(Wp2, PD, C)

    for wq, wkv, wp, wa in ((Wq1, wkv1, wp1, Wa1), (Wq2, wkv2, wp2, Wa2)):
        q2, kvp2 = _qkv(f2, wq, wkv)
        kvn2 = _gather_rows(kvp2, idx2f, C, 64, jnp.int32)
        q3, kvp3 = _qkv(f3, wq, wkv)
        kvn3 = _gather_rows(kvp3, idx3f, C, 64, jnp.int32)
        f2 = _attn(q2, f2, ph2, kvn2, pn2, wp, wa)
        f3 = _attn(q3, f3, ph3, kvn3, pn3, wp, wa)

    wso1 = _pad(W_so[:C], C, 128)
    wso2 = _pad(W_so[C:], C, 128)
    bso = _pad(b_so[None, :], 1, 128)
    pose = _pool(f2, f3, wso1, wso2, bso)[0, :6]

    rot = _aa_to_rot(pose[0:3])
    amat = jnp.zeros((PD, PD), f32).at[0:3, 0:3].set(rot).at[3, 0:3].set(pose[3:6])
    out2d = _proj(ph2, amat)

    kpts_2d_xyz = out2d[:, 0:3].T[None, :, :]
    kpts_3d_xyz = ph3[:, 0:3].T[None, :, :]
    return (kpts_2d_xyz, kpts_3d_xyz)


# speedup vs baseline: 1.1779x; 1.1022x over previous
import functools

import jax
import jax.numpy as jnp
from jax.experimental import pallas as pl
from jax.experimental.pallas import tpu as pltpu
from jax.experimental.pallas import tpu_sc as plsc

N = 1024
C = 512
K = 16
PD = 128
TILE = 128

_INTERPRET = False


def _prep_body(k2_ref, k3_ref, mrows_ref, w2d_ref, b2d_ref, w3d_ref, b3d_ref,
               ph2_ref, ph3_ref, f2_ref, f3_ref):
    lane = jax.lax.broadcasted_iota(jnp.int32, (N, PD), 1)
    x = k2_ref[:, 0:1]
    y = k2_ref[:, 1:2]
    r0 = mrows_ref[0:1, :]
    r1 = mrows_ref[1:2, :]
    r2 = mrows_ref[2:3, :]
    bea = y * r0 + x * r1 + r2
    n2 = jnp.sqrt(jnp.sum(bea * bea, axis=1, keepdims=True))
    p2 = bea / jnp.maximum(n2, 1e-12)
    z = k3_ref[:, 2:3]
    c0 = k3_ref[:, 0:1] / z
    c1 = k3_ref[:, 1:2] / z
    w = jnp.where(lane == 0, c0, jnp.where(lane == 1, c1,
                  jnp.where(lane == 2, 1.0, 0.0)))
    n3 = jnp.sqrt(jnp.sum(w * w, axis=1, keepdims=True))
    p3 = w / jnp.maximum(n3, 1e-12)
    f2_ref[...] = jnp.dot(p2, w2d_ref[...],
                          preferred_element_type=jnp.float32) + b2d_ref[0:1, :]
    f3_ref[...] = jnp.dot(p3, w3d_ref[...],
                          preferred_element_type=jnp.float32) + b3d_ref[0:1, :]
    ph2_ref[...] = jnp.where(lane == 3, 1.0, p2)
    ph3_ref[...] = jnp.where(lane == 3, 1.0, p3)


def _prep(k2, k3, mrows, w2d_pad, b2d, w3d_pad, b3d):
    f32 = jnp.float32
    return pl.pallas_call(
        _prep_body,
        out_shape=(jax.ShapeDtypeStruct((N, PD), f32),
                   jax.ShapeDtypeStruct((N, PD), f32),
                   jax.ShapeDtypeStruct((N, C), f32),
                   jax.ShapeDtypeStruct((N, C), f32)),
        interpret=_INTERPRET,
    )(k2, k3, mrows, w2d_pad, b2d, w3d_pad, b3d)


def _knn_body(ph_ref, phT_ref, idx_ref):
    g = jnp.dot(ph_ref[...], phT_ref[...], preferred_element_type=jnp.float32)
    cols = jax.lax.broadcasted_iota(jnp.int32, (TILE, N), 1)
    packed = ((jax.lax.bitcast_convert_type(g, jnp.int32) & jnp.int32(-1024))
              | (jnp.int32(N - 1) - cols))
    for t in range(K):
        m = jnp.max(packed, axis=1)
        amin = jnp.int32(N - 1) - (m & jnp.int32(N - 1))
        idx_ref[t, :] = amin
        packed = jnp.where(cols == amin[:, None],
                           jnp.int32(-2147483647 - 1), packed)


def _knn(ph, phT):
    return pl.pallas_call(
        _knn_body,
        grid=(N // TILE,),
        in_specs=[pl.BlockSpec((TILE, PD), lambda i: (i, 0)),
                  pl.BlockSpec((PD, N), lambda i: (0, 0))],
        out_specs=pl.BlockSpec((K, TILE), lambda i: (0, i)),
        out_shape=jax.ShapeDtypeStruct((K, N), jnp.int32),
        interpret=_INTERPRET,
    )(ph, phT)


def _rne_bf16_bits(x):
    b = jax.lax.bitcast_convert_type(x, jnp.int32)
    return (b + 0x7FFF + ((b >> 16) & 1)) & jnp.int32(-65536)


def _qkv_body(f_ref, wq_ref, wkv_ref, q_ref, kvp_ref):
    f = f_ref[...].astype(jnp.bfloat16)
    q_ref[...] = jnp.dot(f, wq_ref[...].astype(jnp.bfloat16),
                         preferred_element_type=jnp.float32)
    kv = jnp.dot(f, wkv_ref[...].astype(jnp.bfloat16),
                 preferred_element_type=jnp.float32)
    kb = _rne_bf16_bits(kv[:, 0:C])
    vb = _rne_bf16_bits(kv[:, C:2 * C])
    kvp_ref[...] = kb | ((vb >> 16) & 0xFFFF)


def _qkv(f, wq, wkv):
    return pl.pallas_call(
        _qkv_body,
        out_shape=(jax.ShapeDtypeStruct((N, C), jnp.float32),
                   jax.ShapeDtypeStruct((N, C), jnp.int32)),
        interpret=_INTERPRET,
    )(f, wq, wkv)


def _gather_rows(table, idx_flat, d, chunk, dtype=jnp.float32):
    b_total = idx_flat.shape[0]
    info = plsc.get_sparse_core_info()
    nw = info.num_cores * info.num_subcores
    b_per_w = b_total // nw
    n_chunks = b_per_w // chunk
    mesh = plsc.VectorSubcoreMesh(core_axis_name="c", subcore_axis_name="s")

    @functools.partial(
        pl.kernel,
        out_type=jax.ShapeDtypeStruct((b_total, d), dtype),
        mesh=mesh,
        scratch_types=[
            pltpu.VMEM((b_per_w,), jnp.int32),
            pltpu.VMEM((chunk, d), dtype),
            pltpu.VMEM((chunk, d), dtype),
            pltpu.SemaphoreType.DMA,
            pltpu.SemaphoreType.DMA,
            pltpu.SemaphoreType.DMA,
            pltpu.SemaphoreType.DMA,
        ],
    )
    def kern(table_hbm, idx_hbm, out_hbm, idx_v, buf0, buf1, gs0, gs1, ss0, ss1):
        wid = jax.lax.axis_index("s") * info.num_cores + jax.lax.axis_index("c")
        base = wid * b_per_w
        pltpu.sync_copy(idx_hbm.at[pl.ds(base, b_per_w)], idx_v)
        bufs, gsems, ssems = (buf0, buf1), (gs0, gs1), (ss0, ss1)

        def gstart(ci):
            return pltpu.async_copy(
                table_hbm.at[idx_v.at[pl.ds(ci * chunk, chunk)]],
                bufs[ci % 2], gsems[ci % 2])

        def sstart(ci):
            return pltpu.async_copy(
                bufs[ci % 2], out_hbm.at[pl.ds(base + ci * chunk, chunk)],
                ssems[ci % 2])

        g = {0: gstart(0)}
        s = {}
        for ci in range(n_chunks):
            if ci + 1 < n_chunks:
                if ci - 1 >= 0:
                    s[ci - 1].wait()
                g[ci + 1] = gstart(ci + 1)
            g[ci].wait()
            s[ci] = sstart(ci)
        if n_chunks >= 2:
            s[n_chunks - 2].wait()
        s[n_chunks - 1].wait()

    return kern(table, idx_flat)


def _attn_body(q_ref, f_ref, ph_ref, kvn_ref, pn_ref, wp_ref, wa_ref, out_ref):
    q = q_ref[...]
    ph = ph_ref[...].astype(jnp.bfloat16)
    wp = wp_ref[...].astype(jnp.bfloat16)
    wa = wa_ref[...].astype(jnp.bfloat16)
    s = jnp.zeros(q.shape, jnp.float32)
    acc = jnp.zeros(q.shape, jnp.float32)
    for j in range(K):
        rel = ph - pn_ref[j].astype(jnp.bfloat16)
        pe = jnp.dot(rel, wp, preferred_element_type=jnp.float32)
        kn = jax.lax.bitcast_convert_type(
            kvn_ref[j] & jnp.int32(-65536), jnp.float32)
        t = jnp.maximum(q - kn + pe, 0.0)
        l = jnp.dot(t.astype(jnp.bfloat16), wa,
                    preferred_element_type=jnp.float32)
        e = jnp.exp(l)
        s = s + e
        vn = jax.lax.bitcast_convert_type(kvn_ref[j] << 16, jnp.float32)
        acc = acc + e * (vn + pe)
    out_ref[...] = f_ref[...] + acc / s


ATILE = 256


def _attn(q, f, ph, kvn, pn, wp_pad, wa):
    kvn3 = kvn.reshape(K, N, C)
    pn3 = pn.reshape(K, N, PD)
    return pl.pallas_call(
        _attn_body,
        grid=(N // ATILE,),
        in_specs=[
            pl.BlockSpec((ATILE, C), lambda i: (i, 0)),
            pl.BlockSpec((ATILE, C), lambda i: (i, 0)),
            pl.BlockSpec((ATILE, PD), lambda i: (i, 0)),
            pl.BlockSpec((K, ATILE, C), lambda i: (0, i, 0)),
            pl.BlockSpec((K, ATILE, PD), lambda i: (0, i, 0)),
            pl.BlockSpec((PD, C), lambda i: (0, 0)),
            pl.BlockSpec((C, C), lambda i: (0, 0)),
        ],
        out_specs=pl.BlockSpec((ATILE, C), lambda i: (i, 0)),
        out_shape=jax.ShapeDtypeStruct((N, C), jnp.float32),
        interpret=_INTERPRET,
    )(q, f, ph, kvn3, pn3, wp_pad, wa)


def _pool_body(f2_ref, f3_ref, w1_ref, w2_ref, b_ref, out_ref):
    m2 = jnp.sum(f2_ref[...], axis=0, keepdims=True) * (1.0 / N)
    m3 = jnp.sum(f3_ref[...], axis=0, keepdims=True) * (1.0 / N)
    out_ref[...] = (jnp.dot(m2, w1_ref[...], preferred_element_type=jnp.float32)
                    + jnp.dot(m3, w2_ref[...], preferred_element_type=jnp.float32)
                    + b_ref[...])


def _pool(f2, f3, w1_pad, w2_pad, b_pad):
    return pl.pallas_call(
        _pool_body,
        out_shape=jax.ShapeDtypeStruct((1, 128), jnp.float32),
        interpret=_INTERPRET,
    )(f2, f3, w1_pad, w2_pad, b_pad)


def _proj_body(ph_ref, a_ref, out_ref):
    y = jnp.dot(ph_ref[...], a_ref[...], preferred_element_type=jnp.float32)
    n = jnp.sqrt(jnp.sum(y * y, axis=1, keepdims=True))
    out_ref[...] = y / jnp.maximum(n, 1e-12)


def _proj(ph, amat):
    return pl.pallas_call(
        _proj_body,
        out_shape=jax.ShapeDtypeStruct((N, PD), jnp.float32),
        interpret=_INTERPRET,
    )(ph, amat)


def _aa_to_rot(aa):
    theta = jnp.linalg.norm(aa) + 1e-12
    k = aa / theta
    z = jnp.zeros(())
    kx = jnp.stack([jnp.stack([z, -k[2], k[1]]),
                    jnp.stack([k[2], z, -k[0]]),
                    jnp.stack([-k[1], k[0], z])])
    return (jnp.eye(3) + jnp.sin(theta) * kx
            + (1.0 - jnp.cos(theta)) * (kx @ kx))


def _pad(x, rows, cols):
    return jnp.zeros((rows, cols), jnp.float32).at[:x.shape[0], :x.shape[1]].set(x)


def kernel(kpts_2d_pix, kpts_3d_pts, intrinsics, W2d_pre, b2d_pre, W3d_pre,
           b3d_pre, Wq1, Wk1, Wv1, Wa1, Wq2, Wk2, Wv2, Wa2, Wp1, Wp2,
           W_so, b_so):
    f32 = jnp.float32
    inv_intr = jnp.linalg.inv(intrinsics)
    mrows = _pad(inv_intr.T, 8, PD)
    w2d_pad = _pad(W2d_pre, PD, C)
    w3d_pad = _pad(W3d_pre, PD, C)
    b2d = jnp.broadcast_to(b2d_pre[None, :], (8, C)).astype(f32)
    b3d = jnp.broadcast_to(b3d_pre[None, :], (8, C)).astype(f32)

    ph2, ph3, f2, f3 = _prep(kpts_2d_pix, kpts_3d_pts, mrows,
                             w2d_pad, b2d, w3d_pad, b3d)

    idx2 = _knn(ph2, ph2.T)
    idx3 = _knn(ph3, ph3.T)
    idx2f = idx2.reshape(K * N)
    idx3f = idx3.reshape(K * N)

    pn2 = _gather_rows(ph2, idx2f, PD, 256)
    pn3 = _gather_rows(ph3, idx3f, PD, 256)

    wkv1 = jnp.concatenate([Wk1, Wv1], axis=1)
    wkv2 = jnp.concatenate([Wk2, Wv2], axis=1)
    wp1 = _pad(Wp1, PD, C)
    wp2 = _pad(Wp2, PD, C)

    for wq, wkv, wp, wa in ((Wq1, wkv1, wp1, Wa1), (Wq2, wkv2, wp2, Wa2)):
        q2, kvp2 = _qkv(f2, wq, wkv)
        kvn2 = _gather_rows(kvp2, idx2f, C, 64, jnp.int32)
        q3, kvp3 = _qkv(f3, wq, wkv)
        kvn3 = _gather_rows(kvp3, idx3f, C, 64, jnp.int32)
        f2 = _attn(q2, f2, ph2, kvn2, pn2, wp, wa)
        f3 = _attn(q3, f3, ph3, kvn3, pn3, wp, wa)

    wso1 = _pad(W_so[:C], C, 128)
    wso2 = _pad(W_so[C:], C, 128)
    bso = _pad(b_so[None, :], 1, 128)
    pose = _pool(f2, f3, wso1, wso2, bso)[0, :6]

    rot = _aa_to_rot(pose[0:3])
    amat = jnp.zeros((PD, PD), f32).at[0:3, 0:3].set(rot).at[3, 0:3].set(pose[3:6])
    out2d = _proj(ph2, amat)

    kpts_2d_xyz = out2d[:, 0:3].T[None, :, :]
    kpts_3d_xyz = ph3[:, 0:3].T[None, :, :]
    return (kpts_2d_xyz, kpts_3d_xyz)
